# Initial kernel scaffold; baseline (speedup 1.0000x reference)
import jax
import jax.numpy as jnp
from jax.experimental import pallas as pl


def kernel(data, x, edge_index, W1, a_src1, a_dst1, b1, W2, a_src2, a_dst2, b2):
    raise NotImplementedError("write your pallas kernel here")


# speedup vs baseline: 1.0754x; 1.0754x over previous
import functools

import jax
import jax.numpy as jnp
from jax.experimental import pallas as pl
from jax.experimental.pallas import tpu as pltpu

N = 10000
E = 320000
NFEAT = 128
NHID = 8
HEADS = 8
NCLASS = 16

_BN = 1000


def _tc1_body(x_ref, w1_ref, as_ref, ad_ref, h_ref, at_ref, bt_ref):
    h = jnp.dot(x_ref[...], w1_ref[...], preferred_element_type=jnp.float32)
    h_ref[...] = h
    at_ref[...] = jnp.dot(h, as_ref[...], preferred_element_type=jnp.float32)
    bt_ref[...] = jnp.dot(h, ad_ref[...], preferred_element_type=jnp.float32)


def _tc1(x, W1, As, Ad):
    D = W1.shape[1]
    return pl.pallas_call(
        _tc1_body,
        grid=(N // _BN,),
        in_specs=[
            pl.BlockSpec((_BN, NFEAT), lambda i: (i, 0)),
            pl.BlockSpec((NFEAT, D), lambda i: (0, 0)),
            pl.BlockSpec((D, HEADS), lambda i: (0, 0)),
            pl.BlockSpec((D, HEADS), lambda i: (0, 0)),
        ],
        out_specs=[
            pl.BlockSpec((_BN, D), lambda i: (i, 0)),
            pl.BlockSpec((_BN, HEADS), lambda i: (i, 0)),
            pl.BlockSpec((_BN, HEADS), lambda i: (i, 0)),
        ],
        out_shape=[
            jax.ShapeDtypeStruct((N, D), jnp.float32),
            jax.ShapeDtypeStruct((N, HEADS), jnp.float32),
            jax.ShapeDtypeStruct((N, HEADS), jnp.float32),
        ],
    )(x, W1, As, Ad)


def _tc2_body(p_ref, b1_ref, w2_ref, as_ref, ad_ref, h2_ref, at_ref, bt_ref):
    o = p_ref[0] + p_ref[1] + b1_ref[...]
    o = jnp.where(o > 0.0, o, jnp.exp(o) - 1.0)
    h2 = jnp.dot(o, w2_ref[...], preferred_element_type=jnp.float32)
    h2_ref[...] = h2
    at_ref[...] = jnp.dot(h2, as_ref[...], preferred_element_type=jnp.float32)
    bt_ref[...] = jnp.dot(h2, ad_ref[...], preferred_element_type=jnp.float32)


def _tc2(parts, b1, W2, As2, Ad2):
    D1 = HEADS * NHID
    return pl.pallas_call(
        _tc2_body,
        grid=(N // _BN,),
        in_specs=[
            pl.BlockSpec((2, _BN, D1), lambda i: (0, i, 0)),
            pl.BlockSpec((1, D1), lambda i: (0, 0)),
            pl.BlockSpec((D1, NCLASS), lambda i: (0, 0)),
            pl.BlockSpec((NCLASS, HEADS), lambda i: (0, 0)),
            pl.BlockSpec((NCLASS, HEADS), lambda i: (0, 0)),
        ],
        out_specs=[
            pl.BlockSpec((_BN, NCLASS), lambda i: (i, 0)),
            pl.BlockSpec((_BN, HEADS), lambda i: (i, 0)),
            pl.BlockSpec((_BN, HEADS), lambda i: (i, 0)),
        ],
        out_shape=[
            jax.ShapeDtypeStruct((N, NCLASS), jnp.float32),
            jax.ShapeDtypeStruct((N, HEADS), jnp.float32),
            jax.ShapeDtypeStruct((N, HEADS), jnp.float32),
        ],
    )(parts, b1, W2, As2, Ad2)


def _tc3_body(p_ref, b2_ref, o_ref):
    o_ref[...] = p_ref[0] + p_ref[1] + b2_ref[...]


def _tc3(parts, b2):
    return pl.pallas_call(
        _tc3_body,
        grid=(N // _BN,),
        in_specs=[
            pl.BlockSpec((2, _BN, NCLASS), lambda i: (0, i, 0)),
            pl.BlockSpec((1, NCLASS), lambda i: (0, 0)),
        ],
        out_specs=pl.BlockSpec((_BN, NCLASS), lambda i: (i, 0)),
        out_shape=jax.ShapeDtypeStruct((N, NCLASS), jnp.float32),
    )(parts, b2)


def _edge_phase_jax(h, at, bt, edge_index, heads, ch):
    src = edge_index[0]
    dst = edge_index[1]
    alpha = at[src] + bt[dst]
    alpha = alpha[:, :heads]
    alpha = jax.nn.leaky_relu(alpha, negative_slope=0.2)
    amax = jax.ops.segment_max(alpha, dst, num_segments=N)
    alpha = jnp.exp(alpha - amax[dst])
    denom = jax.ops.segment_sum(alpha, dst, num_segments=N)
    alpha = alpha / (denom[dst] + 1e-16)
    msg = h.reshape(N, heads, ch)[src] * alpha[:, :, None]
    out = jax.ops.segment_sum(msg, dst, num_segments=N)
    return out.reshape(N, heads * ch)


def kernel(data, x, edge_index, W1, a_src1, a_dst1, b1, W2, a_src2, a_dst2, b2):
    eye = jnp.eye(HEADS, dtype=jnp.float32)
    As1 = (eye[:, None, :] * a_src1[:, :, None]).reshape(HEADS * NHID, HEADS)
    Ad1 = (eye[:, None, :] * a_dst1[:, :, None]).reshape(HEADS * NHID, HEADS)
    As2 = jnp.pad(a_src2.reshape(NCLASS, 1), ((0, 0), (0, HEADS - 1)))
    Ad2 = jnp.pad(a_dst2.reshape(NCLASS, 1), ((0, 0), (0, HEADS - 1)))

    h1, at1, bt1 = _tc1(x, W1, As1, Ad1)
    o1 = _edge_phase_jax(h1, at1, bt1, edge_index, HEADS, NHID)
    parts1 = jnp.stack([o1, jnp.zeros_like(o1)])
    h2, at2, bt2 = _tc2(parts1, b1.reshape(1, -1), W2, As2, Ad2)
    o2 = _edge_phase_jax(h2, at2, bt2, edge_index, 1, NCLASS)
    parts2 = jnp.stack([o2, jnp.zeros_like(o2)])
    return _tc3(parts2, b2.reshape(1, -1))


# speedup vs baseline: 23.8214x; 22.1505x over previous
import functools

import jax
import jax.numpy as jnp
from jax import lax
from jax.experimental import pallas as pl
from jax.experimental.pallas import tpu as pltpu
from jax.experimental.pallas import tpu_sc as plsc

N = 10000
E = 320000
NFEAT = 128
NHID = 8
HEADS = 8
NCLASS = 16

_BN = 1000


def _tc1_body(x_ref, w1_ref, as_ref, ad_ref, h_ref, at_ref, bt_ref):
    h = jnp.dot(x_ref[...], w1_ref[...], preferred_element_type=jnp.float32)
    h_ref[...] = h
    at_ref[...] = jnp.dot(h, as_ref[...], preferred_element_type=jnp.float32)
    bt_ref[...] = jnp.dot(h, ad_ref[...], preferred_element_type=jnp.float32)


def _tc1(x, W1, As, Ad):
    D = W1.shape[1]
    return pl.pallas_call(
        _tc1_body,
        grid=(N // _BN,),
        in_specs=[
            pl.BlockSpec((_BN, NFEAT), lambda i: (i, 0)),
            pl.BlockSpec((NFEAT, D), lambda i: (0, 0)),
            pl.BlockSpec((D, 16), lambda i: (0, 0)),
            pl.BlockSpec((D, 16), lambda i: (0, 0)),
        ],
        out_specs=[
            pl.BlockSpec((_BN, D), lambda i: (i, 0)),
            pl.BlockSpec((_BN, 16), lambda i: (i, 0)),
            pl.BlockSpec((_BN, 16), lambda i: (i, 0)),
        ],
        out_shape=[
            jax.ShapeDtypeStruct((N, D), jnp.float32),
            jax.ShapeDtypeStruct((N, 16), jnp.float32),
            jax.ShapeDtypeStruct((N, 16), jnp.float32),
        ],
    )(x, W1, As, Ad)


def _tc2_body(p_ref, b1_ref, w2_ref, as_ref, ad_ref, h2_ref, at_ref, bt_ref):
    o = p_ref[0] + p_ref[1] + b1_ref[...]
    o = jnp.where(o > 0.0, o, jnp.exp(o) - 1.0)
    h2 = jnp.dot(o, w2_ref[...], preferred_element_type=jnp.float32)
    h2_ref[...] = h2
    at_ref[...] = jnp.dot(h2, as_ref[...], preferred_element_type=jnp.float32)
    bt_ref[...] = jnp.dot(h2, ad_ref[...], preferred_element_type=jnp.float32)


def _tc2(parts, b1, W2, As2, Ad2):
    D1 = HEADS * NHID
    return pl.pallas_call(
        _tc2_body,
        grid=(N // _BN,),
        in_specs=[
            pl.BlockSpec((2, _BN, D1), lambda i: (0, i, 0)),
            pl.BlockSpec((1, D1), lambda i: (0, 0)),
            pl.BlockSpec((D1, NCLASS), lambda i: (0, 0)),
            pl.BlockSpec((NCLASS, 16), lambda i: (0, 0)),
            pl.BlockSpec((NCLASS, 16), lambda i: (0, 0)),
        ],
        out_specs=[
            pl.BlockSpec((_BN, NCLASS), lambda i: (i, 0)),
            pl.BlockSpec((_BN, 16), lambda i: (i, 0)),
            pl.BlockSpec((_BN, 16), lambda i: (i, 0)),
        ],
        out_shape=[
            jax.ShapeDtypeStruct((N, NCLASS), jnp.float32),
            jax.ShapeDtypeStruct((N, 16), jnp.float32),
            jax.ShapeDtypeStruct((N, 16), jnp.float32),
        ],
    )(parts, b1, W2, As2, Ad2)


def _tc3_body(p_ref, b2_ref, o_ref):
    o_ref[...] = p_ref[0] + p_ref[1] + b2_ref[...]


def _tc3(parts, b2):
    return pl.pallas_call(
        _tc3_body,
        grid=(N // _BN,),
        in_specs=[
            pl.BlockSpec((2, _BN, NCLASS), lambda i: (0, i, 0)),
            pl.BlockSpec((1, NCLASS), lambda i: (0, 0)),
        ],
        out_specs=pl.BlockSpec((_BN, NCLASS), lambda i: (i, 0)),
        out_shape=jax.ShapeDtypeStruct((N, NCLASS), jnp.float32),
    )(parts, b2)


_CH = 640
_CR = 5
_NCHUNK = E // _CH
_NB = _NCHUNK // 2
_RPT = 624


def _make_sc_edge(D, C):
    Q = D // 16
    csh = C.bit_length() - 1
    mesh = plsc.VectorSubcoreMesh(core_axis_name="c", subcore_axis_name="s")

    @functools.partial(
        pl.kernel,
        out_type=jax.ShapeDtypeStruct((2, N, D), jnp.float32),
        mesh=mesh,
        compiler_params=pltpu.CompilerParams(use_tc_tiling_on_sc=False),
        scratch_types=[
            pltpu.VMEM_SHARED((N, 16), jnp.float32),
            pltpu.VMEM_SHARED((N, D), jnp.float32),
            pltpu.VMEM((_CR, 128), jnp.int32),
            pltpu.VMEM((_CR, 128), jnp.int32),
            pltpu.VMEM((_CH, 16), jnp.float32),
            pltpu.VMEM((_CH, 16), jnp.float32),
            pltpu.VMEM((_CH, 16), jnp.float32),
            pltpu.VMEM((_CH, D), jnp.float32),
        ],
    )
    def sc(ei_ref, h_ref, at_ref, bt_ref, z_ref, z16_ref, out_ref,
           den_s, acc_s, sidx, didx, ga, gb, gd, hg):
        cid = lax.axis_index("c")
        tid = lax.axis_index("s")
        r0 = tid * _RPT
        rows = pl.ds(r0, _RPT)
        tail = pl.ds(16 * _RPT, N - 16 * _RPT)

        def stage(sl):
            pltpu.sync_copy(z16_ref.at[sl], den_s.at[sl])
            pltpu.sync_copy(z_ref.at[sl], acc_s.at[sl])

        stage(rows)

        @pl.when(tid == 15)
        def _():
            stage(tail)

        plsc.subcore_barrier()

        iota = lax.iota(jnp.int32, 16)
        dnums = lax.GatherDimensionNumbers(
            offset_dims=(), collapsed_slice_dims=(0,), start_index_map=(0,))

        def bcast(v, pat):
            return lax.gather(v, pat[:, None], dimension_numbers=dnums,
                              slice_sizes=(1,),
                              mode=lax.GatherScatterMode.PROMISE_IN_BOUNDS)

        pats = [(q * 16 + iota) >> csh for q in range(Q)]

        def load_idx(cc):
            pltpu.sync_copy(ei_ref.at[0, cc], sidx)
            pltpu.sync_copy(ei_ref.at[1, cc], didx)

        def alpha_body(i, _):
            v = ga[i] + gb[i]
            v = jnp.where(v >= 0.0, v, 0.2 * v)
            ga[i] = jnp.exp(v)
            return 0

        def pass_a(k, _):
            cc = tid + 16 * k

            @pl.when(cc < _NCHUNK)
            def _():
                load_idx(cc)
                for j in range(_CR):
                    sl = pl.ds(j * 128, 128)
                    pltpu.sync_copy(at_ref.at[sidx.at[j]], ga.at[sl])
                    pltpu.sync_copy(bt_ref.at[didx.at[j]], gb.at[sl])
                lax.fori_loop(0, _CH, alpha_body, 0)
                for j in range(_CR):
                    sl = pl.ds(j * 128, 128)
                    pltpu.sync_copy(ga.at[sl], den_s.at[didx.at[j]], add=True)
            return 0

        lax.fori_loop(0, (_NCHUNK + 15) // 16, pass_a, 0)
        plsc.subcore_barrier()

        def weight_body(i, _):
            v = ga[i] + gb[i]
            v = jnp.where(v >= 0.0, v, 0.2 * v)
            w = jnp.exp(v) / gd[i]
            ga[i] = w
            for q in range(Q):
                sl = pl.ds(q * 16, 16)
                hg[i, sl] = hg[i, sl] * bcast(w, pats[q])
            return 0

        def pass_b(k, _):
            cb = tid + 16 * k

            @pl.when(cb < _NB)
            def _():
                cc = cid * _NB + cb
                load_idx(cc)
                for j in range(_CR):
                    sl = pl.ds(j * 128, 128)
                    pltpu.sync_copy(at_ref.at[sidx.at[j]], ga.at[sl])
                    pltpu.sync_copy(bt_ref.at[didx.at[j]], gb.at[sl])
                    pltpu.sync_copy(den_s.at[didx.at[j]], gd.at[sl])
                    pltpu.sync_copy(h_ref.at[sidx.at[j]], hg.at[sl])
                lax.fori_loop(0, _CH, weight_body, 0)
                for j in range(_CR):
                    sl = pl.ds(j * 128, 128)
                    pltpu.sync_copy(hg.at[sl], acc_s.at[didx.at[j]], add=True)
            return 0

        lax.fori_loop(0, (_NB + 15) // 16, pass_b, 0)
        plsc.subcore_barrier()
        pltpu.sync_copy(acc_s.at[rows], out_ref.at[cid, rows])

        @pl.when(tid == 15)
        def _():
            pltpu.sync_copy(acc_s.at[tail], out_ref.at[cid, tail])

    return sc


_sc_edge1 = _make_sc_edge(HEADS * NHID, NHID)
_sc_edge2 = _make_sc_edge(NCLASS, NCLASS)


def _edge_phase_jax(h, at, bt, edge_index, heads, ch):
    src = edge_index[0]
    dst = edge_index[1]
    alpha = at[src] + bt[dst]
    alpha = alpha[:, :heads]
    alpha = jax.nn.leaky_relu(alpha, negative_slope=0.2)
    amax = jax.ops.segment_max(alpha, dst, num_segments=N)
    alpha = jnp.exp(alpha - amax[dst])
    denom = jax.ops.segment_sum(alpha, dst, num_segments=N)
    alpha = alpha / (denom[dst] + 1e-16)
    msg = h.reshape(N, heads, ch)[src] * alpha[:, :, None]
    out = jax.ops.segment_sum(msg, dst, num_segments=N)
    return out.reshape(N, heads * ch)


def kernel(data, x, edge_index, W1, a_src1, a_dst1, b1, W2, a_src2, a_dst2, b2):
    eye = jnp.eye(HEADS, dtype=jnp.float32)
    As1 = jnp.pad((eye[:, None, :] * a_src1[:, :, None]).reshape(HEADS * NHID, HEADS),
                  ((0, 0), (0, 16 - HEADS)))
    Ad1 = jnp.pad((eye[:, None, :] * a_dst1[:, :, None]).reshape(HEADS * NHID, HEADS),
                  ((0, 0), (0, 16 - HEADS)))
    As2 = jnp.pad(a_src2.reshape(NCLASS, 1), ((0, 0), (0, 15)))
    Ad2 = jnp.pad(a_dst2.reshape(NCLASS, 1), ((0, 0), (0, 15)))

    ei4 = edge_index.reshape(2, _NCHUNK, _CR, 128)
    z64 = jnp.zeros((N, HEADS * NHID), jnp.float32)
    z16 = jnp.zeros((N, NCLASS), jnp.float32)
    z16 = jnp.zeros((N, 16), jnp.float32)

    h1, at1, bt1 = _tc1(x, W1, As1, Ad1)
    parts1 = _sc_edge1(ei4, h1, at1, bt1, z64, z16)
    h2, at2, bt2 = _tc2(parts1, b1.reshape(1, -1), W2, As2, Ad2)
    parts2 = _sc_edge2(ei4, h2, at2, bt2, z16, z16)
    return _tc3(parts2, b2.reshape(1, -1))


# speedup vs baseline: 31.5310x; 1.3236x over previous
import functools

import jax
import jax.numpy as jnp
from jax import lax
from jax.experimental import pallas as pl
from jax.experimental.pallas import tpu as pltpu
from jax.experimental.pallas import tpu_sc as plsc

N = 10000
E = 320000
NFEAT = 128
NHID = 8
HEADS = 8
NCLASS = 16

_BN = 1000


def _tc1_body(x_ref, w1_ref, as_ref, ad_ref, h_ref, at_ref, bt_ref):
    h = jnp.dot(x_ref[...], w1_ref[...], preferred_element_type=jnp.float32)
    h_ref[...] = h
    at_ref[...] = jnp.dot(h, as_ref[...], preferred_element_type=jnp.float32)
    bt_ref[...] = jnp.dot(h, ad_ref[...], preferred_element_type=jnp.float32)


def _tc1(x, W1, As, Ad):
    D = W1.shape[1]
    return pl.pallas_call(
        _tc1_body,
        grid=(N // _BN,),
        in_specs=[
            pl.BlockSpec((_BN, NFEAT), lambda i: (i, 0)),
            pl.BlockSpec((NFEAT, D), lambda i: (0, 0)),
            pl.BlockSpec((D, 16), lambda i: (0, 0)),
            pl.BlockSpec((D, 16), lambda i: (0, 0)),
        ],
        out_specs=[
            pl.BlockSpec((_BN, D), lambda i: (i, 0)),
            pl.BlockSpec((_BN, 16), lambda i: (i, 0)),
            pl.BlockSpec((_BN, 16), lambda i: (i, 0)),
        ],
        out_shape=[
            jax.ShapeDtypeStruct((N, D), jnp.float32),
            jax.ShapeDtypeStruct((N, 16), jnp.float32),
            jax.ShapeDtypeStruct((N, 16), jnp.float32),
        ],
    )(x, W1, As, Ad)


def _tc2_body(p_ref, b1_ref, w2_ref, as_ref, ad_ref, h2_ref, at_ref, bt_ref):
    o = p_ref[0] + p_ref[1] + b1_ref[...]
    o = jnp.where(o > 0.0, o, jnp.exp(o) - 1.0)
    h2 = jnp.dot(o, w2_ref[...], preferred_element_type=jnp.float32)
    h2_ref[...] = h2
    at_ref[...] = jnp.dot(h2, as_ref[...], preferred_element_type=jnp.float32)
    bt_ref[...] = jnp.dot(h2, ad_ref[...], preferred_element_type=jnp.float32)


def _tc2(parts, b1, W2, As2, Ad2):
    D1 = HEADS * NHID
    return pl.pallas_call(
        _tc2_body,
        grid=(N // _BN,),
        in_specs=[
            pl.BlockSpec((2, _BN, D1), lambda i: (0, i, 0)),
            pl.BlockSpec((1, D1), lambda i: (0, 0)),
            pl.BlockSpec((D1, NCLASS), lambda i: (0, 0)),
            pl.BlockSpec((NCLASS, 16), lambda i: (0, 0)),
            pl.BlockSpec((NCLASS, 16), lambda i: (0, 0)),
        ],
        out_specs=[
            pl.BlockSpec((_BN, NCLASS), lambda i: (i, 0)),
            pl.BlockSpec((_BN, 16), lambda i: (i, 0)),
            pl.BlockSpec((_BN, 16), lambda i: (i, 0)),
        ],
        out_shape=[
            jax.ShapeDtypeStruct((N, NCLASS), jnp.float32),
            jax.ShapeDtypeStruct((N, 16), jnp.float32),
            jax.ShapeDtypeStruct((N, 16), jnp.float32),
        ],
    )(parts, b1, W2, As2, Ad2)


def _tc3_body(p_ref, b2_ref, o_ref):
    o_ref[...] = p_ref[0] + p_ref[1] + b2_ref[...]


def _tc3(parts, b2):
    return pl.pallas_call(
        _tc3_body,
        grid=(N // _BN,),
        in_specs=[
            pl.BlockSpec((2, _BN, NCLASS), lambda i: (0, i, 0)),
            pl.BlockSpec((1, NCLASS), lambda i: (0, 0)),
        ],
        out_specs=pl.BlockSpec((_BN, NCLASS), lambda i: (i, 0)),
        out_shape=jax.ShapeDtypeStruct((N, NCLASS), jnp.float32),
    )(parts, b2)


_CH = 640
_CR = 5
_NCHUNK = E // _CH
_NB = _NCHUNK // 2
_RPT = 624


def _make_sc_edge(D, C):
    Q = D // 16
    csh = C.bit_length() - 1
    mesh = plsc.VectorSubcoreMesh(core_axis_name="c", subcore_axis_name="s")

    @functools.partial(
        pl.kernel,
        out_type=jax.ShapeDtypeStruct((2, N, D), jnp.float32),
        mesh=mesh,
        compiler_params=pltpu.CompilerParams(use_tc_tiling_on_sc=False),
        scratch_types=[
            pltpu.VMEM_SHARED((N, 16), jnp.float32),
            pltpu.VMEM_SHARED((N, D), jnp.float32),
            pltpu.VMEM((_CR, 128), jnp.int32),
            pltpu.VMEM((_CR, 128), jnp.int32),
            pltpu.VMEM((_CH, 16), jnp.float32),
            pltpu.VMEM((_CH, 16), jnp.float32),
            pltpu.VMEM((_CH, 16), jnp.float32),
            pltpu.VMEM((_CH, D), jnp.float32),
            pltpu.SemaphoreType.DMA((4,)),
        ],
    )
    def sc(ei_ref, h_ref, at_ref, bt_ref, z_ref, z16_ref, out_ref,
           den_s, acc_s, sidx, didx, ga, gb, gd, hg, sems):
        cid = lax.axis_index("c")
        tid = lax.axis_index("s")
        r0 = tid * _RPT
        rows = pl.ds(r0, _RPT)
        tail = pl.ds(16 * _RPT, N - 16 * _RPT)

        def stage(sl):
            pltpu.sync_copy(z16_ref.at[sl], den_s.at[sl])
            pltpu.sync_copy(z_ref.at[sl], acc_s.at[sl])

        stage(rows)

        @pl.when(tid == 15)
        def _():
            stage(tail)

        plsc.subcore_barrier()

        iota = lax.iota(jnp.int32, 16)
        dnums = lax.GatherDimensionNumbers(
            offset_dims=(), collapsed_slice_dims=(0,), start_index_map=(0,))

        def bcast(v, pat):
            return lax.gather(v, pat[:, None], dimension_numbers=dnums,
                              slice_sizes=(1,),
                              mode=lax.GatherScatterMode.PROMISE_IN_BOUNDS)

        pats = [(q * 16 + iota) >> csh for q in range(Q)]

        def load_idx(cc):
            d0 = pltpu.async_copy(ei_ref.at[0, cc], sidx, sems.at[0])
            d1 = pltpu.async_copy(ei_ref.at[1, cc], didx, sems.at[1])
            d0.wait()
            d1.wait()

        def alpha_body(i, _):
            v = ga[i] + gb[i]
            v = jnp.where(v >= 0.0, v, 0.2 * v)
            ga[i] = jnp.exp(v)
            return 0

        def pass_a(k, _):
            cc = tid + 16 * k

            @pl.when(cc < _NCHUNK)
            def _():
                load_idx(cc)
                for j in range(_CR):
                    sl = pl.ds(j * 128, 128)
                    cur = [pltpu.async_copy(at_ref.at[sidx.at[j]], ga.at[sl], sems.at[0]),
                           pltpu.async_copy(bt_ref.at[didx.at[j]], gb.at[sl], sems.at[1])]
                    for d in cur:
                        d.wait()
                lax.fori_loop(0, _CH, alpha_body, 0)
                for j in range(_CR):
                    sl = pl.ds(j * 128, 128)
                    pltpu.sync_copy(ga.at[sl], den_s.at[didx.at[j]], add=True)
            return 0

        lax.fori_loop(0, (_NCHUNK + 15) // 16, pass_a, 0)
        plsc.subcore_barrier()

        def weight_body(i, _):
            v = ga[i] + gb[i]
            v = jnp.where(v >= 0.0, v, 0.2 * v)
            w = jnp.exp(v) / gd[i]
            ga[i] = w
            for q in range(Q):
                sl = pl.ds(q * 16, 16)
                hg[i, sl] = hg[i, sl] * bcast(w, pats[q])
            return 0

        def pass_b(k, _):
            cb = tid + 16 * k

            @pl.when(cb < _NB)
            def _():
                cc = cid * _NB + cb
                load_idx(cc)
                for j in range(_CR):
                    sl = pl.ds(j * 128, 128)
                    cur = [pltpu.async_copy(at_ref.at[sidx.at[j]], ga.at[sl], sems.at[0]),
                           pltpu.async_copy(bt_ref.at[didx.at[j]], gb.at[sl], sems.at[1]),
                           pltpu.async_copy(den_s.at[didx.at[j]], gd.at[sl], sems.at[2]),
                           pltpu.async_copy(h_ref.at[sidx.at[j]], hg.at[sl], sems.at[3])]
                    for d in cur:
                        d.wait()
                lax.fori_loop(0, _CH, weight_body, 0)
                for j in range(_CR):
                    sl = pl.ds(j * 128, 128)
                    pltpu.sync_copy(hg.at[sl], acc_s.at[didx.at[j]], add=True)
            return 0

        lax.fori_loop(0, (_NB + 15) // 16, pass_b, 0)
        plsc.subcore_barrier()
        pltpu.sync_copy(acc_s.at[rows], out_ref.at[cid, rows])

        @pl.when(tid == 15)
        def _():
            pltpu.sync_copy(acc_s.at[tail], out_ref.at[cid, tail])

    return sc


_sc_edge1 = _make_sc_edge(HEADS * NHID, NHID)
_sc_edge2 = _make_sc_edge(NCLASS, NCLASS)


def _edge_phase_jax(h, at, bt, edge_index, heads, ch):
    src = edge_index[0]
    dst = edge_index[1]
    alpha = at[src] + bt[dst]
    alpha = alpha[:, :heads]
    alpha = jax.nn.leaky_relu(alpha, negative_slope=0.2)
    amax = jax.ops.segment_max(alpha, dst, num_segments=N)
    alpha = jnp.exp(alpha - amax[dst])
    denom = jax.ops.segment_sum(alpha, dst, num_segments=N)
    alpha = alpha / (denom[dst] + 1e-16)
    msg = h.reshape(N, heads, ch)[src] * alpha[:, :, None]
    out = jax.ops.segment_sum(msg, dst, num_segments=N)
    return out.reshape(N, heads * ch)


def kernel(data, x, edge_index, W1, a_src1, a_dst1, b1, W2, a_src2, a_dst2, b2):
    eye = jnp.eye(HEADS, dtype=jnp.float32)
    As1 = jnp.pad((eye[:, None, :] * a_src1[:, :, None]).reshape(HEADS * NHID, HEADS),
                  ((0, 0), (0, 16 - HEADS)))
    Ad1 = jnp.pad((eye[:, None, :] * a_dst1[:, :, None]).reshape(HEADS * NHID, HEADS),
                  ((0, 0), (0, 16 - HEADS)))
    As2 = jnp.pad(a_src2.reshape(NCLASS, 1), ((0, 0), (0, 15)))
    Ad2 = jnp.pad(a_dst2.reshape(NCLASS, 1), ((0, 0), (0, 15)))

    ei4 = edge_index.reshape(2, _NCHUNK, _CR, 128)
    z64 = jnp.zeros((N, HEADS * NHID), jnp.float32)
    z16 = jnp.zeros((N, NCLASS), jnp.float32)
    z16 = jnp.zeros((N, 16), jnp.float32)

    h1, at1, bt1 = _tc1(x, W1, As1, Ad1)
    parts1 = _sc_edge1(ei4, h1, at1, bt1, z64, z16)
    h2, at2, bt2 = _tc2(parts1, b1.reshape(1, -1), W2, As2, Ad2)
    parts2 = _sc_edge2(ei4, h2, at2, bt2, z16, z16)
    return _tc3(parts2, b2.reshape(1, -1))


# speedup vs baseline: 34.3747x; 1.0902x over previous
import functools

import jax
import jax.numpy as jnp
from jax import lax
from jax.experimental import pallas as pl
from jax.experimental.pallas import tpu as pltpu
from jax.experimental.pallas import tpu_sc as plsc

N = 10000
E = 320000
NFEAT = 128
NHID = 8
HEADS = 8
NCLASS = 16

_BN = 1000


def _tc1_body(x_ref, w1_ref, as_ref, ad_ref, h_ref, at_ref, bt_ref):
    h = jnp.dot(x_ref[...], w1_ref[...], preferred_element_type=jnp.float32)
    h_ref[...] = h
    at_ref[...] = jnp.dot(h, as_ref[...], preferred_element_type=jnp.float32)
    bt_ref[...] = jnp.dot(h, ad_ref[...], preferred_element_type=jnp.float32)


def _tc1(x, W1, As, Ad):
    D = W1.shape[1]
    return pl.pallas_call(
        _tc1_body,
        grid=(N // _BN,),
        in_specs=[
            pl.BlockSpec((_BN, NFEAT), lambda i: (i, 0)),
            pl.BlockSpec((NFEAT, D), lambda i: (0, 0)),
            pl.BlockSpec((D, 16), lambda i: (0, 0)),
            pl.BlockSpec((D, 16), lambda i: (0, 0)),
        ],
        out_specs=[
            pl.BlockSpec((_BN, D), lambda i: (i, 0)),
            pl.BlockSpec((_BN, 16), lambda i: (i, 0)),
            pl.BlockSpec((_BN, 16), lambda i: (i, 0)),
        ],
        out_shape=[
            jax.ShapeDtypeStruct((N, D), jnp.float32),
            jax.ShapeDtypeStruct((N, 16), jnp.float32),
            jax.ShapeDtypeStruct((N, 16), jnp.float32),
        ],
    )(x, W1, As, Ad)


def _tc2_body(p_ref, b1_ref, w2_ref, as_ref, ad_ref, h2_ref, at_ref, bt_ref):
    o = p_ref[0] + p_ref[1] + b1_ref[...]
    o = jnp.where(o > 0.0, o, jnp.exp(o) - 1.0)
    h2 = jnp.dot(o, w2_ref[...], preferred_element_type=jnp.float32)
    h2_ref[...] = h2
    at_ref[...] = jnp.dot(h2, as_ref[...], preferred_element_type=jnp.float32)
    bt_ref[...] = jnp.dot(h2, ad_ref[...], preferred_element_type=jnp.float32)


def _tc2(parts, b1, W2, As2, Ad2):
    D1 = HEADS * NHID
    return pl.pallas_call(
        _tc2_body,
        grid=(N // _BN,),
        in_specs=[
            pl.BlockSpec((2, _BN, D1), lambda i: (0, i, 0)),
            pl.BlockSpec((1, D1), lambda i: (0, 0)),
            pl.BlockSpec((D1, NCLASS), lambda i: (0, 0)),
            pl.BlockSpec((NCLASS, 16), lambda i: (0, 0)),
            pl.BlockSpec((NCLASS, 16), lambda i: (0, 0)),
        ],
        out_specs=[
            pl.BlockSpec((_BN, NCLASS), lambda i: (i, 0)),
            pl.BlockSpec((_BN, 16), lambda i: (i, 0)),
            pl.BlockSpec((_BN, 16), lambda i: (i, 0)),
        ],
        out_shape=[
            jax.ShapeDtypeStruct((N, NCLASS), jnp.float32),
            jax.ShapeDtypeStruct((N, 16), jnp.float32),
            jax.ShapeDtypeStruct((N, 16), jnp.float32),
        ],
    )(parts, b1, W2, As2, Ad2)


def _tc3_body(p_ref, b2_ref, o_ref):
    o_ref[...] = p_ref[0] + p_ref[1] + b2_ref[...]


def _tc3(parts, b2):
    return pl.pallas_call(
        _tc3_body,
        grid=(N // _BN,),
        in_specs=[
            pl.BlockSpec((2, _BN, NCLASS), lambda i: (0, i, 0)),
            pl.BlockSpec((1, NCLASS), lambda i: (0, 0)),
        ],
        out_specs=pl.BlockSpec((_BN, NCLASS), lambda i: (i, 0)),
        out_shape=jax.ShapeDtypeStruct((N, NCLASS), jnp.float32),
    )(parts, b2)


_CH = 640
_CR = 5
_NCHUNK = E // _CH
_NB = _NCHUNK // 2
_RPT = 624


def _make_sc_edge(D, C):
    Q = D // 16
    csh = C.bit_length() - 1
    mesh = plsc.VectorSubcoreMesh(core_axis_name="c", subcore_axis_name="s")

    @functools.partial(
        pl.kernel,
        out_type=jax.ShapeDtypeStruct((2, N, D), jnp.float32),
        mesh=mesh,
        compiler_params=pltpu.CompilerParams(use_tc_tiling_on_sc=False),
        scratch_types=[
            pltpu.VMEM_SHARED((N, 16), jnp.float32),
            pltpu.VMEM_SHARED((N, D), jnp.float32),
            pltpu.VMEM((_CR, 128), jnp.int32),
            pltpu.VMEM((_CR, 128), jnp.int32),
            pltpu.VMEM((_CH, 16), jnp.float32),
            pltpu.VMEM((_CH, 16), jnp.float32),
            pltpu.VMEM((_CH, 16), jnp.float32),
            pltpu.VMEM((_CH, D), jnp.float32),
            pltpu.SemaphoreType.DMA((4, 2)),
        ],
    )
    def sc(ei_ref, h_ref, at_ref, bt_ref, z_ref, z16_ref, out_ref,
           den_s, acc_s, sidx, didx, ga, gb, gd, hg, sems):
        cid = lax.axis_index("c")
        tid = lax.axis_index("s")
        r0 = tid * _RPT
        rows = pl.ds(r0, _RPT)
        tail = pl.ds(16 * _RPT, N - 16 * _RPT)

        def stage(sl):
            pltpu.sync_copy(z16_ref.at[sl], den_s.at[sl])
            pltpu.sync_copy(z_ref.at[sl], acc_s.at[sl])

        stage(rows)

        @pl.when(tid == 15)
        def _():
            stage(tail)

        plsc.subcore_barrier()

        iota = lax.iota(jnp.int32, 16)
        dnums = lax.GatherDimensionNumbers(
            offset_dims=(), collapsed_slice_dims=(0,), start_index_map=(0,))

        def bcast(v, pat):
            return lax.gather(v, pat[:, None], dimension_numbers=dnums,
                              slice_sizes=(1,),
                              mode=lax.GatherScatterMode.PROMISE_IN_BOUNDS)

        pats = [(q * 16 + iota) >> csh for q in range(Q)]

        def load_idx(cc):
            d0 = pltpu.async_copy(ei_ref.at[0, cc], sidx, sems.at[0, 0])
            d1 = pltpu.async_copy(ei_ref.at[1, cc], didx, sems.at[1, 0])
            d0.wait()
            d1.wait()

        def alpha_body(i, _):
            v = ga[i] + gb[i]
            v = jnp.where(v >= 0.0, v, 0.2 * v)
            ga[i] = jnp.exp(v)
            return 0

        def pass_a(k, _):
            cc = tid + 16 * k

            @pl.when(cc < _NCHUNK)
            def _():
                load_idx(cc)
                prev = []
                for j in range(_CR):
                    sl = pl.ds(j * 128, 128)
                    p = j & 1
                    cur = [pltpu.async_copy(at_ref.at[sidx.at[j]], ga.at[sl], sems.at[0, p]),
                           pltpu.async_copy(bt_ref.at[didx.at[j]], gb.at[sl], sems.at[1, p])]
                    for d in prev:
                        d.wait()
                    prev = cur
                for d in prev:
                    d.wait()
                lax.fori_loop(0, _CH, alpha_body, 0)
                for j in range(_CR):
                    sl = pl.ds(j * 128, 128)
                    pltpu.sync_copy(ga.at[sl], den_s.at[didx.at[j]], add=True)
            return 0

        lax.fori_loop(0, (_NCHUNK + 15) // 16, pass_a, 0)
        plsc.subcore_barrier()

        def weight_body(i, _):
            v = ga[i] + gb[i]
            v = jnp.where(v >= 0.0, v, 0.2 * v)
            w = jnp.exp(v) / gd[i]
            ga[i] = w
            for q in range(Q):
                sl = pl.ds(q * 16, 16)
                hg[i, sl] = hg[i, sl] * bcast(w, pats[q])
            return 0

        def pass_b(k, _):
            cb = tid + 16 * k

            @pl.when(cb < _NB)
            def _():
                cc = cid * _NB + cb
                load_idx(cc)
                prev = []
                for j in range(_CR):
                    sl = pl.ds(j * 128, 128)
                    p = j & 1
                    cur = [pltpu.async_copy(at_ref.at[sidx.at[j]], ga.at[sl], sems.at[0, p]),
                           pltpu.async_copy(bt_ref.at[didx.at[j]], gb.at[sl], sems.at[1, p]),
                           pltpu.async_copy(den_s.at[didx.at[j]], gd.at[sl], sems.at[2, p]),
                           pltpu.async_copy(h_ref.at[sidx.at[j]], hg.at[sl], sems.at[3, p])]
                    for d in prev:
                        d.wait()
                    prev = cur
                for d in prev:
                    d.wait()
                lax.fori_loop(0, _CH, weight_body, 0)
                for j in range(_CR):
                    sl = pl.ds(j * 128, 128)
                    pltpu.sync_copy(hg.at[sl], acc_s.at[didx.at[j]], add=True)
            return 0

        lax.fori_loop(0, (_NB + 15) // 16, pass_b, 0)
        plsc.subcore_barrier()
        pltpu.sync_copy(acc_s.at[rows], out_ref.at[cid, rows])

        @pl.when(tid == 15)
        def _():
            pltpu.sync_copy(acc_s.at[tail], out_ref.at[cid, tail])

    return sc


_sc_edge1 = _make_sc_edge(HEADS * NHID, NHID)
_sc_edge2 = _make_sc_edge(NCLASS, NCLASS)


def _edge_phase_jax(h, at, bt, edge_index, heads, ch):
    src = edge_index[0]
    dst = edge_index[1]
    alpha = at[src] + bt[dst]
    alpha = alpha[:, :heads]
    alpha = jax.nn.leaky_relu(alpha, negative_slope=0.2)
    amax = jax.ops.segment_max(alpha, dst, num_segments=N)
    alpha = jnp.exp(alpha - amax[dst])
    denom = jax.ops.segment_sum(alpha, dst, num_segments=N)
    alpha = alpha / (denom[dst] + 1e-16)
    msg = h.reshape(N, heads, ch)[src] * alpha[:, :, None]
    out = jax.ops.segment_sum(msg, dst, num_segments=N)
    return out.reshape(N, heads * ch)


def kernel(data, x, edge_index, W1, a_src1, a_dst1, b1, W2, a_src2, a_dst2, b2):
    eye = jnp.eye(HEADS, dtype=jnp.float32)
    As1 = jnp.pad((eye[:, None, :] * a_src1[:, :, None]).reshape(HEADS * NHID, HEADS),
                  ((0, 0), (0, 16 - HEADS)))
    Ad1 = jnp.pad((eye[:, None, :] * a_dst1[:, :, None]).reshape(HEADS * NHID, HEADS),
                  ((0, 0), (0, 16 - HEADS)))
    As2 = jnp.pad(a_src2.reshape(NCLASS, 1), ((0, 0), (0, 15)))
    Ad2 = jnp.pad(a_dst2.reshape(NCLASS, 1), ((0, 0), (0, 15)))

    ei4 = edge_index.reshape(2, _NCHUNK, _CR, 128)
    z64 = jnp.zeros((N, HEADS * NHID), jnp.float32)
    z16 = jnp.zeros((N, NCLASS), jnp.float32)
    z16 = jnp.zeros((N, 16), jnp.float32)

    h1, at1, bt1 = _tc1(x, W1, As1, Ad1)
    parts1 = _sc_edge1(ei4, h1, at1, bt1, z64, z16)
    h2, at2, bt2 = _tc2(parts1, b1.reshape(1, -1), W2, As2, Ad2)
    parts2 = _sc_edge2(ei4, h2, at2, bt2, z16, z16)
    return _tc3(parts2, b2.reshape(1, -1))


# speedup vs baseline: 36.4352x; 1.0599x over previous
import functools

import jax
import jax.numpy as jnp
from jax import lax
from jax.experimental import pallas as pl
from jax.experimental.pallas import tpu as pltpu
from jax.experimental.pallas import tpu_sc as plsc

N = 10000
E = 320000
NFEAT = 128
NHID = 8
HEADS = 8
NCLASS = 16

_BN = 1000


def _tc1_body(x_ref, w1_ref, as_ref, ad_ref, h_ref, at_ref, bt_ref):
    h = jnp.dot(x_ref[...], w1_ref[...], preferred_element_type=jnp.float32)
    h_ref[...] = h
    at_ref[...] = jnp.dot(h, as_ref[...], preferred_element_type=jnp.float32)
    bt_ref[...] = jnp.dot(h, ad_ref[...], preferred_element_type=jnp.float32)


def _tc1(x, W1, As, Ad):
    D = W1.shape[1]
    return pl.pallas_call(
        _tc1_body,
        grid=(N // _BN,),
        in_specs=[
            pl.BlockSpec((_BN, NFEAT), lambda i: (i, 0)),
            pl.BlockSpec((NFEAT, D), lambda i: (0, 0)),
            pl.BlockSpec((D, 16), lambda i: (0, 0)),
            pl.BlockSpec((D, 16), lambda i: (0, 0)),
        ],
        out_specs=[
            pl.BlockSpec((_BN, D), lambda i: (i, 0)),
            pl.BlockSpec((_BN, 16), lambda i: (i, 0)),
            pl.BlockSpec((_BN, 16), lambda i: (i, 0)),
        ],
        out_shape=[
            jax.ShapeDtypeStruct((N, D), jnp.float32),
            jax.ShapeDtypeStruct((N, 16), jnp.float32),
            jax.ShapeDtypeStruct((N, 16), jnp.float32),
        ],
    )(x, W1, As, Ad)


def _tc2_body(p_ref, b1_ref, w2_ref, as_ref, ad_ref, h2_ref, at_ref, bt_ref):
    o = p_ref[0] + p_ref[1] + b1_ref[...]
    o = jnp.where(o > 0.0, o, jnp.exp(o) - 1.0)
    h2 = jnp.dot(o, w2_ref[...], preferred_element_type=jnp.float32)
    h2_ref[...] = h2
    at_ref[...] = jnp.dot(h2, as_ref[...], preferred_element_type=jnp.float32)
    bt_ref[...] = jnp.dot(h2, ad_ref[...], preferred_element_type=jnp.float32)


def _tc2(parts, b1, W2, As2, Ad2):
    D1 = HEADS * NHID
    return pl.pallas_call(
        _tc2_body,
        grid=(N // _BN,),
        in_specs=[
            pl.BlockSpec((2, _BN, D1), lambda i: (0, i, 0)),
            pl.BlockSpec((1, D1), lambda i: (0, 0)),
            pl.BlockSpec((D1, NCLASS), lambda i: (0, 0)),
            pl.BlockSpec((NCLASS, 16), lambda i: (0, 0)),
            pl.BlockSpec((NCLASS, 16), lambda i: (0, 0)),
        ],
        out_specs=[
            pl.BlockSpec((_BN, NCLASS), lambda i: (i, 0)),
            pl.BlockSpec((_BN, 16), lambda i: (i, 0)),
            pl.BlockSpec((_BN, 16), lambda i: (i, 0)),
        ],
        out_shape=[
            jax.ShapeDtypeStruct((N, NCLASS), jnp.float32),
            jax.ShapeDtypeStruct((N, 16), jnp.float32),
            jax.ShapeDtypeStruct((N, 16), jnp.float32),
        ],
    )(parts, b1, W2, As2, Ad2)


def _tc3_body(p_ref, b2_ref, o_ref):
    o_ref[...] = p_ref[0] + p_ref[1] + b2_ref[...]


def _tc3(parts, b2):
    return pl.pallas_call(
        _tc3_body,
        grid=(N // _BN,),
        in_specs=[
            pl.BlockSpec((2, _BN, NCLASS), lambda i: (0, i, 0)),
            pl.BlockSpec((1, NCLASS), lambda i: (0, 0)),
        ],
        out_specs=pl.BlockSpec((_BN, NCLASS), lambda i: (i, 0)),
        out_shape=jax.ShapeDtypeStruct((N, NCLASS), jnp.float32),
    )(parts, b2)


_CH = 640
_CR = 5
_NCHUNK = E // _CH
_NB = _NCHUNK // 2
_RPT = 624


def _make_sc_edge(D, C):
    Q = D // 16
    csh = C.bit_length() - 1
    mesh = plsc.VectorSubcoreMesh(core_axis_name="c", subcore_axis_name="s")

    @functools.partial(
        pl.kernel,
        out_type=jax.ShapeDtypeStruct((2, N, D), jnp.float32),
        mesh=mesh,
        compiler_params=pltpu.CompilerParams(use_tc_tiling_on_sc=False),
        scratch_types=[
            pltpu.VMEM_SHARED((N, 16), jnp.float32),
            pltpu.VMEM_SHARED((N, D), jnp.float32),
            pltpu.VMEM((_CR, 128), jnp.int32),
            pltpu.VMEM((_CR, 128), jnp.int32),
            pltpu.VMEM((_CH, 16), jnp.float32),
            pltpu.VMEM((_CH, 16), jnp.float32),
            pltpu.VMEM((_CH, 16), jnp.float32),
            pltpu.VMEM((_CH, D), jnp.float32),
            pltpu.SemaphoreType.DMA((4, _CR)),
            pltpu.SemaphoreType.DMA((2,)),
        ],
    )
    def sc(ei_ref, h_ref, at_ref, bt_ref, z_ref, z16_ref, out_ref,
           den_s, acc_s, sidx, didx, ga, gb, gd, hg, sems, ssems):
        cid = lax.axis_index("c")
        tid = lax.axis_index("s")
        r0 = tid * _RPT
        rows = pl.ds(r0, _RPT)
        tail = pl.ds(16 * _RPT, N - 16 * _RPT)

        def stage(sl):
            pltpu.sync_copy(z16_ref.at[sl], den_s.at[sl])
            pltpu.sync_copy(z_ref.at[sl], acc_s.at[sl])

        stage(rows)

        @pl.when(tid == 15)
        def _():
            stage(tail)

        plsc.subcore_barrier()

        iota = lax.iota(jnp.int32, 16)
        dnums = lax.GatherDimensionNumbers(
            offset_dims=(), collapsed_slice_dims=(0,), start_index_map=(0,))

        def bcast(v, pat):
            return lax.gather(v, pat[:, None], dimension_numbers=dnums,
                              slice_sizes=(1,),
                              mode=lax.GatherScatterMode.PROMISE_IN_BOUNDS)

        pats = [(q * 16 + iota) >> csh for q in range(Q)]

        def load_idx(cc):
            d0 = pltpu.async_copy(ei_ref.at[0, cc], sidx, sems.at[0, 0])
            d1 = pltpu.async_copy(ei_ref.at[1, cc], didx, sems.at[1, 0])
            d0.wait()
            d1.wait()

        def alpha_body(i, _):
            v = ga[i] + gb[i]
            v = jnp.where(v >= 0.0, v, 0.2 * v)
            ga[i] = jnp.exp(v)
            return 0

        def pass_a(k, _):
            cc = tid + 16 * k

            @pl.when(cc < _NCHUNK)
            def _():
                load_idx(cc)
                ds_ = []
                for j in range(_CR):
                    sl = pl.ds(j * 128, 128)
                    ds_.append(pltpu.async_copy(at_ref.at[sidx.at[j]], ga.at[sl], sems.at[0, j]))
                    ds_.append(pltpu.async_copy(bt_ref.at[didx.at[j]], gb.at[sl], sems.at[1, j]))
                for d in ds_:
                    d.wait()
                lax.fori_loop(0, _CH, alpha_body, 0)
                prev = []
                for j in range(_CR):
                    sl = pl.ds(j * 128, 128)
                    cur = [pltpu.async_copy(ga.at[sl], den_s.at[didx.at[j]], ssems.at[j & 1], add=True)]
                    for d in prev:
                        d.wait()
                    prev = cur
                for d in prev:
                    d.wait()
            return 0

        lax.fori_loop(0, (_NCHUNK + 15) // 16, pass_a, 0)
        plsc.subcore_barrier()

        def weight_body(i, _):
            v = ga[i] + gb[i]
            v = jnp.where(v >= 0.0, v, 0.2 * v)
            w = jnp.exp(v) / gd[i]
            ga[i] = w
            for q in range(Q):
                sl = pl.ds(q * 16, 16)
                hg[i, sl] = hg[i, sl] * bcast(w, pats[q])
            return 0

        def pass_b(k, _):
            cb = tid + 16 * k

            @pl.when(cb < _NB)
            def _():
                cc = cid * _NB + cb
                load_idx(cc)
                ds_ = []
                for j in range(_CR):
                    sl = pl.ds(j * 128, 128)
                    ds_.append(pltpu.async_copy(at_ref.at[sidx.at[j]], ga.at[sl], sems.at[0, j]))
                    ds_.append(pltpu.async_copy(bt_ref.at[didx.at[j]], gb.at[sl], sems.at[1, j]))
                    ds_.append(pltpu.async_copy(den_s.at[didx.at[j]], gd.at[sl], sems.at[2, j]))
                    ds_.append(pltpu.async_copy(h_ref.at[sidx.at[j]], hg.at[sl], sems.at[3, j]))
                for d in ds_:
                    d.wait()
                lax.fori_loop(0, _CH, weight_body, 0)
                prev = []
                for j in range(_CR):
                    sl = pl.ds(j * 128, 128)
                    cur = [pltpu.async_copy(hg.at[sl], acc_s.at[didx.at[j]], ssems.at[j & 1], add=True)]
                    for d in prev:
                        d.wait()
                    prev = cur
                for d in prev:
                    d.wait()
            return 0

        lax.fori_loop(0, (_NB + 15) // 16, pass_b, 0)
        plsc.subcore_barrier()
        pltpu.sync_copy(acc_s.at[rows], out_ref.at[cid, rows])

        @pl.when(tid == 15)
        def _():
            pltpu.sync_copy(acc_s.at[tail], out_ref.at[cid, tail])

    return sc


_sc_edge1 = _make_sc_edge(HEADS * NHID, NHID)
_sc_edge2 = _make_sc_edge(NCLASS, NCLASS)


def _edge_phase_jax(h, at, bt, edge_index, heads, ch):
    src = edge_index[0]
    dst = edge_index[1]
    alpha = at[src] + bt[dst]
    alpha = alpha[:, :heads]
    alpha = jax.nn.leaky_relu(alpha, negative_slope=0.2)
    amax = jax.ops.segment_max(alpha, dst, num_segments=N)
    alpha = jnp.exp(alpha - amax[dst])
    denom = jax.ops.segment_sum(alpha, dst, num_segments=N)
    alpha = alpha / (denom[dst] + 1e-16)
    msg = h.reshape(N, heads, ch)[src] * alpha[:, :, None]
    out = jax.ops.segment_sum(msg, dst, num_segments=N)
    return out.reshape(N, heads * ch)


def kernel(data, x, edge_index, W1, a_src1, a_dst1, b1, W2, a_src2, a_dst2, b2):
    eye = jnp.eye(HEADS, dtype=jnp.float32)
    As1 = jnp.pad((eye[:, None, :] * a_src1[:, :, None]).reshape(HEADS * NHID, HEADS),
                  ((0, 0), (0, 16 - HEADS)))
    Ad1 = jnp.pad((eye[:, None, :] * a_dst1[:, :, None]).reshape(HEADS * NHID, HEADS),
                  ((0, 0), (0, 16 - HEADS)))
    As2 = jnp.pad(a_src2.reshape(NCLASS, 1), ((0, 0), (0, 15)))
    Ad2 = jnp.pad(a_dst2.reshape(NCLASS, 1), ((0, 0), (0, 15)))

    ei4 = edge_index.reshape(2, _NCHUNK, _CR, 128)
    z64 = jnp.zeros((N, HEADS * NHID), jnp.float32)
    z16 = jnp.zeros((N, NCLASS), jnp.float32)
    z16 = jnp.zeros((N, 16), jnp.float32)

    h1, at1, bt1 = _tc1(x, W1, As1, Ad1)
    parts1 = _sc_edge1(ei4, h1, at1, bt1, z64, z16)
    h2, at2, bt2 = _tc2(parts1, b1.reshape(1, -1), W2, As2, Ad2)
    parts2 = _sc_edge2(ei4, h2, at2, bt2, z16, z16)
    return _tc3(parts2, b2.reshape(1, -1))


# speedup vs baseline: 42.2252x; 1.1589x over previous
import functools

import jax
import jax.numpy as jnp
from jax import lax
from jax.experimental import pallas as pl
from jax.experimental.pallas import tpu as pltpu
from jax.experimental.pallas import tpu_sc as plsc

N = 10000
E = 320000
NFEAT = 128
NHID = 8
HEADS = 8
NCLASS = 16

_BN = 1000


def _tc1_body(x_ref, w1_ref, as_ref, ad_ref, h_ref, at_ref, bt_ref):
    h = jnp.dot(x_ref[...], w1_ref[...], preferred_element_type=jnp.float32)
    h_ref[...] = h
    at_ref[...] = jnp.dot(h, as_ref[...], preferred_element_type=jnp.float32)
    bt_ref[...] = jnp.dot(h, ad_ref[...], preferred_element_type=jnp.float32)


def _tc1(x, W1, As, Ad):
    D = W1.shape[1]
    return pl.pallas_call(
        _tc1_body,
        grid=(N // _BN,),
        in_specs=[
            pl.BlockSpec((_BN, NFEAT), lambda i: (i, 0)),
            pl.BlockSpec((NFEAT, D), lambda i: (0, 0)),
            pl.BlockSpec((D, 16), lambda i: (0, 0)),
            pl.BlockSpec((D, 16), lambda i: (0, 0)),
        ],
        out_specs=[
            pl.BlockSpec((_BN, D), lambda i: (i, 0)),
            pl.BlockSpec((_BN, 16), lambda i: (i, 0)),
            pl.BlockSpec((_BN, 16), lambda i: (i, 0)),
        ],
        out_shape=[
            jax.ShapeDtypeStruct((N, D), jnp.float32),
            jax.ShapeDtypeStruct((N, 16), jnp.float32),
            jax.ShapeDtypeStruct((N, 16), jnp.float32),
        ],
    )(x, W1, As, Ad)


def _tc2_body(p_ref, b1_ref, w2_ref, as_ref, ad_ref, h2_ref, at_ref, bt_ref):
    o = p_ref[0] + p_ref[1] + b1_ref[...]
    o = jnp.where(o > 0.0, o, jnp.exp(o) - 1.0)
    h2 = jnp.dot(o, w2_ref[...], preferred_element_type=jnp.float32)
    h2_ref[...] = h2
    at_ref[...] = jnp.dot(h2, as_ref[...], preferred_element_type=jnp.float32)
    bt_ref[...] = jnp.dot(h2, ad_ref[...], preferred_element_type=jnp.float32)


def _tc2(parts, b1, W2, As2, Ad2):
    D1 = HEADS * NHID
    return pl.pallas_call(
        _tc2_body,
        grid=(N // _BN,),
        in_specs=[
            pl.BlockSpec((2, _BN, D1), lambda i: (0, i, 0)),
            pl.BlockSpec((1, D1), lambda i: (0, 0)),
            pl.BlockSpec((D1, NCLASS), lambda i: (0, 0)),
            pl.BlockSpec((NCLASS, 16), lambda i: (0, 0)),
            pl.BlockSpec((NCLASS, 16), lambda i: (0, 0)),
        ],
        out_specs=[
            pl.BlockSpec((_BN, NCLASS), lambda i: (i, 0)),
            pl.BlockSpec((_BN, 16), lambda i: (i, 0)),
            pl.BlockSpec((_BN, 16), lambda i: (i, 0)),
        ],
        out_shape=[
            jax.ShapeDtypeStruct((N, NCLASS), jnp.float32),
            jax.ShapeDtypeStruct((N, 16), jnp.float32),
            jax.ShapeDtypeStruct((N, 16), jnp.float32),
        ],
    )(parts, b1, W2, As2, Ad2)


def _tc3_body(p_ref, b2_ref, o_ref):
    o_ref[...] = p_ref[0] + p_ref[1] + b2_ref[...]


def _tc3(parts, b2):
    return pl.pallas_call(
        _tc3_body,
        grid=(N // _BN,),
        in_specs=[
            pl.BlockSpec((2, _BN, NCLASS), lambda i: (0, i, 0)),
            pl.BlockSpec((1, NCLASS), lambda i: (0, 0)),
        ],
        out_specs=pl.BlockSpec((_BN, NCLASS), lambda i: (i, 0)),
        out_shape=jax.ShapeDtypeStruct((N, NCLASS), jnp.float32),
    )(parts, b2)


_CH = 640
_CR = 5
_NCHUNK = E // _CH
_NB = _NCHUNK // 2
_RPT = 624


def _make_sc_edge(D, C):
    Q = D // 16
    csh = C.bit_length() - 1
    mesh = plsc.VectorSubcoreMesh(core_axis_name="c", subcore_axis_name="s")

    @functools.partial(
        pl.kernel,
        out_type=(jax.ShapeDtypeStruct((2, N, D), jnp.float32),
                  jax.ShapeDtypeStruct((E, 16), jnp.float32)),
        mesh=mesh,
        compiler_params=pltpu.CompilerParams(use_tc_tiling_on_sc=False),
        scratch_types=[
            pltpu.VMEM_SHARED((N, 16), jnp.float32),
            pltpu.VMEM_SHARED((N, D), jnp.float32),
            pltpu.VMEM((_CR, 128), jnp.int32),
            pltpu.VMEM((_CR, 128), jnp.int32),
            pltpu.VMEM((_CH, 16), jnp.float32),
            pltpu.VMEM((_CH, 16), jnp.float32),
            pltpu.VMEM((_CH, 16), jnp.float32),
            pltpu.VMEM((_CH, D), jnp.float32),
            pltpu.SemaphoreType.DMA((4, _CR)),
            pltpu.SemaphoreType.DMA((2,)),
        ],
    )
    def sc(ei_ref, h_ref, at_ref, bt_ref, z_ref, z16_ref, out_ref, al_ref,
           den_s, acc_s, sidx, didx, ga, gb, gd, hg, sems, ssems):
        cid = lax.axis_index("c")
        tid = lax.axis_index("s")
        r0 = tid * _RPT
        rows = pl.ds(r0, _RPT)
        tail = pl.ds(16 * _RPT, N - 16 * _RPT)

        def stage(sl):
            pltpu.sync_copy(z16_ref.at[sl], den_s.at[sl])
            pltpu.sync_copy(z_ref.at[sl], acc_s.at[sl])

        stage(rows)

        @pl.when(tid == 15)
        def _():
            stage(tail)

        plsc.subcore_barrier()

        iota = lax.iota(jnp.int32, 16)
        dnums = lax.GatherDimensionNumbers(
            offset_dims=(), collapsed_slice_dims=(0,), start_index_map=(0,))

        def bcast(v, pat):
            return lax.gather(v, pat[:, None], dimension_numbers=dnums,
                              slice_sizes=(1,),
                              mode=lax.GatherScatterMode.PROMISE_IN_BOUNDS)

        pats = [(q * 16 + iota) >> csh for q in range(Q)]

        def load_idx(cc):
            d0 = pltpu.async_copy(ei_ref.at[0, cc], sidx, sems.at[0, 0])
            d1 = pltpu.async_copy(ei_ref.at[1, cc], didx, sems.at[1, 0])
            d0.wait()
            d1.wait()

        def alpha_body(i, _):
            v = ga[i] + gb[i]
            v = jnp.where(v >= 0.0, v, 0.2 * v)
            ga[i] = jnp.exp(v)
            return 0

        def pass_a(k, _):
            cc = tid + 16 * k

            @pl.when(cc < _NCHUNK)
            def _():
                load_idx(cc)
                ds_ = []
                for j in range(_CR):
                    sl = pl.ds(j * 128, 128)
                    ds_.append(pltpu.async_copy(at_ref.at[sidx.at[j]], ga.at[sl], sems.at[0, j]))
                    ds_.append(pltpu.async_copy(bt_ref.at[didx.at[j]], gb.at[sl], sems.at[1, j]))
                for d in ds_:
                    d.wait()
                lax.fori_loop(0, _CH, alpha_body, 0)
                mine = (cc >= cid * _NB) & (cc < (cid + 1) * _NB)

                @pl.when(mine)
                def _():
                    pltpu.async_copy(ga, al_ref.at[pl.ds(cc * _CH, _CH)],
                                     sems.at[2, 0]).wait()
                prev = []
                for j in range(_CR):
                    sl = pl.ds(j * 128, 128)
                    cur = [pltpu.async_copy(ga.at[sl], den_s.at[didx.at[j]], ssems.at[j & 1], add=True)]
                    for d in prev:
                        d.wait()
                    prev = cur
                for d in prev:
                    d.wait()
            return 0

        lax.fori_loop(0, (_NCHUNK + 15) // 16, pass_a, 0)
        plsc.subcore_barrier()

        def weight_body(i, _):
            w = ga[i] / gd[i]
            for q in range(Q):
                sl = pl.ds(q * 16, 16)
                hg[i, sl] = hg[i, sl] * bcast(w, pats[q])
            return 0

        def pass_b(k, _):
            cb = tid + 16 * k

            @pl.when(cb < _NB)
            def _():
                cc = cid * _NB + cb
                load_idx(cc)
                ds_ = [pltpu.async_copy(al_ref.at[pl.ds(cc * _CH, _CH)], ga,
                                        sems.at[0, 0])]
                for j in range(_CR):
                    sl = pl.ds(j * 128, 128)
                    ds_.append(pltpu.async_copy(den_s.at[didx.at[j]], gd.at[sl], sems.at[2, j]))
                    ds_.append(pltpu.async_copy(h_ref.at[sidx.at[j]], hg.at[sl], sems.at[3, j]))
                for d in ds_:
                    d.wait()
                lax.fori_loop(0, _CH, weight_body, 0)
                prev = []
                for j in range(_CR):
                    sl = pl.ds(j * 128, 128)
                    cur = [pltpu.async_copy(hg.at[sl], acc_s.at[didx.at[j]], ssems.at[j & 1], add=True)]
                    for d in prev:
                        d.wait()
                    prev = cur
                for d in prev:
                    d.wait()
            return 0

        lax.fori_loop(0, (_NB + 15) // 16, pass_b, 0)
        plsc.subcore_barrier()
        pltpu.sync_copy(acc_s.at[rows], out_ref.at[cid, rows])

        @pl.when(tid == 15)
        def _():
            pltpu.sync_copy(acc_s.at[tail], out_ref.at[cid, tail])

    return sc


_sc_edge1 = _make_sc_edge(HEADS * NHID, NHID)
_sc_edge2 = _make_sc_edge(NCLASS, NCLASS)


def _edge_phase_jax(h, at, bt, edge_index, heads, ch):
    src = edge_index[0]
    dst = edge_index[1]
    alpha = at[src] + bt[dst]
    alpha = alpha[:, :heads]
    alpha = jax.nn.leaky_relu(alpha, negative_slope=0.2)
    amax = jax.ops.segment_max(alpha, dst, num_segments=N)
    alpha = jnp.exp(alpha - amax[dst])
    denom = jax.ops.segment_sum(alpha, dst, num_segments=N)
    alpha = alpha / (denom[dst] + 1e-16)
    msg = h.reshape(N, heads, ch)[src] * alpha[:, :, None]
    out = jax.ops.segment_sum(msg, dst, num_segments=N)
    return out.reshape(N, heads * ch)


def kernel(data, x, edge_index, W1, a_src1, a_dst1, b1, W2, a_src2, a_dst2, b2):
    eye = jnp.eye(HEADS, dtype=jnp.float32)
    As1 = jnp.pad((eye[:, None, :] * a_src1[:, :, None]).reshape(HEADS * NHID, HEADS),
                  ((0, 0), (0, 16 - HEADS)))
    Ad1 = jnp.pad((eye[:, None, :] * a_dst1[:, :, None]).reshape(HEADS * NHID, HEADS),
                  ((0, 0), (0, 16 - HEADS)))
    As2 = jnp.pad(a_src2.reshape(NCLASS, 1), ((0, 0), (0, 15)))
    Ad2 = jnp.pad(a_dst2.reshape(NCLASS, 1), ((0, 0), (0, 15)))

    ei4 = edge_index.reshape(2, _NCHUNK, _CR, 128)
    z64 = jnp.zeros((N, HEADS * NHID), jnp.float32)
    z16 = jnp.zeros((N, NCLASS), jnp.float32)
    z16 = jnp.zeros((N, 16), jnp.float32)

    h1, at1, bt1 = _tc1(x, W1, As1, Ad1)
    parts1, _ = _sc_edge1(ei4, h1, at1, bt1, z64, z16)
    h2, at2, bt2 = _tc2(parts1, b1.reshape(1, -1), W2, As2, Ad2)
    parts2, _ = _sc_edge2(ei4, h2, at2, bt2, z16, z16)
    return _tc3(parts2, b2.reshape(1, -1))


# speedup vs baseline: 89.2187x; 2.1129x over previous
import functools

import jax
import jax.numpy as jnp
from jax import lax
from jax.experimental import pallas as pl
from jax.experimental.pallas import tpu as pltpu
from jax.experimental.pallas import tpu_sc as plsc

N = 10000
E = 320000
NFEAT = 128
NHID = 8
HEADS = 8
NCLASS = 16

_BN = 1000


def _tc1_body(x_ref, w1_ref, as_ref, ad_ref, h_ref, at_ref, bt_ref):
    h = jnp.dot(x_ref[...], w1_ref[...], preferred_element_type=jnp.float32)
    h_ref[...] = h
    at_ref[...] = jnp.dot(h, as_ref[...], preferred_element_type=jnp.float32)
    bt_ref[...] = jnp.dot(h, ad_ref[...], preferred_element_type=jnp.float32)


def _tc1(x, W1, As, Ad):
    D = W1.shape[1]
    return pl.pallas_call(
        _tc1_body,
        grid=(N // _BN,),
        in_specs=[
            pl.BlockSpec((_BN, NFEAT), lambda i: (i, 0)),
            pl.BlockSpec((NFEAT, D), lambda i: (0, 0)),
            pl.BlockSpec((D, 16), lambda i: (0, 0)),
            pl.BlockSpec((D, 16), lambda i: (0, 0)),
        ],
        out_specs=[
            pl.BlockSpec((_BN, D), lambda i: (i, 0)),
            pl.BlockSpec((_BN, 16), lambda i: (i, 0)),
            pl.BlockSpec((_BN, 16), lambda i: (i, 0)),
        ],
        out_shape=[
            jax.ShapeDtypeStruct((N, D), jnp.float32),
            jax.ShapeDtypeStruct((N, 16), jnp.float32),
            jax.ShapeDtypeStruct((N, 16), jnp.float32),
        ],
    )(x, W1, As, Ad)


def _tc2_body(p_ref, b1_ref, w2_ref, as_ref, ad_ref, h2_ref, at_ref, bt_ref):
    o = p_ref[0] + p_ref[1] + b1_ref[...]
    o = jnp.where(o > 0.0, o, jnp.exp(o) - 1.0)
    h2 = jnp.dot(o, w2_ref[...], preferred_element_type=jnp.float32)
    h2_ref[...] = h2
    at_ref[...] = jnp.dot(h2, as_ref[...], preferred_element_type=jnp.float32)
    bt_ref[...] = jnp.dot(h2, ad_ref[...], preferred_element_type=jnp.float32)


def _tc2(parts, b1, W2, As2, Ad2):
    D1 = HEADS * NHID
    return pl.pallas_call(
        _tc2_body,
        grid=(N // _BN,),
        in_specs=[
            pl.BlockSpec((2, _BN, D1), lambda i: (0, i, 0)),
            pl.BlockSpec((1, D1), lambda i: (0, 0)),
            pl.BlockSpec((D1, NCLASS), lambda i: (0, 0)),
            pl.BlockSpec((NCLASS, 16), lambda i: (0, 0)),
            pl.BlockSpec((NCLASS, 16), lambda i: (0, 0)),
        ],
        out_specs=[
            pl.BlockSpec((_BN, NCLASS), lambda i: (i, 0)),
            pl.BlockSpec((_BN, 16), lambda i: (i, 0)),
            pl.BlockSpec((_BN, 16), lambda i: (i, 0)),
        ],
        out_shape=[
            jax.ShapeDtypeStruct((N, NCLASS), jnp.float32),
            jax.ShapeDtypeStruct((N, 16), jnp.float32),
            jax.ShapeDtypeStruct((N, 16), jnp.float32),
        ],
    )(parts, b1, W2, As2, Ad2)


def _tc3_body(p_ref, b2_ref, o_ref):
    o_ref[...] = p_ref[0] + p_ref[1] + b2_ref[...]


def _tc3(parts, b2):
    return pl.pallas_call(
        _tc3_body,
        grid=(N // _BN,),
        in_specs=[
            pl.BlockSpec((2, _BN, NCLASS), lambda i: (0, i, 0)),
            pl.BlockSpec((1, NCLASS), lambda i: (0, 0)),
        ],
        out_specs=pl.BlockSpec((_BN, NCLASS), lambda i: (i, 0)),
        out_shape=jax.ShapeDtypeStruct((N, NCLASS), jnp.float32),
    )(parts, b2)


_CH = 640
_CR = 5
_NCHUNK = E // _CH
_NB = _NCHUNK // 2
_RPT = 624


def _make_sc_edge(D, C):
    Q = D // 16
    csh = C.bit_length() - 1
    mesh = plsc.VectorSubcoreMesh(core_axis_name="c", subcore_axis_name="s")

    @functools.partial(
        pl.kernel,
        out_type=(jax.ShapeDtypeStruct((2, N, D), jnp.float32),
                  jax.ShapeDtypeStruct((E, 16), jnp.float32)),
        mesh=mesh,
        compiler_params=pltpu.CompilerParams(use_tc_tiling_on_sc=False),
        scratch_types=[
            pltpu.VMEM_SHARED((N, 16), jnp.float32),
            pltpu.VMEM_SHARED((N, D), jnp.float32),
            pltpu.VMEM((_CR, 128), jnp.int32),
            pltpu.VMEM((_CR, 128), jnp.int32),
            pltpu.VMEM((_CH, 16), jnp.float32),
            pltpu.VMEM((_CH, 16), jnp.float32),
            pltpu.VMEM((_CH, 16), jnp.float32),
            pltpu.VMEM((_CH, D), jnp.float32),
            pltpu.SemaphoreType.DMA((4, _CR)),
            pltpu.SemaphoreType.DMA((2,)),
        ],
    )
    def sc(ei_ref, h_ref, at_ref, bt_ref, z_ref, z16_ref, out_ref, al_ref,
           den_s, acc_s, sidx, didx, ga, gb, gd, hg, sems, ssems):
        cid = lax.axis_index("c")
        tid = lax.axis_index("s")
        r0 = tid * _RPT
        rows = pl.ds(r0, _RPT)
        tail = pl.ds(16 * _RPT, N - 16 * _RPT)

        def stage(sl):
            pltpu.sync_copy(z16_ref.at[sl], den_s.at[sl])
            pltpu.sync_copy(z_ref.at[sl], acc_s.at[sl])

        stage(rows)

        @pl.when(tid == 15)
        def _():
            stage(tail)

        plsc.subcore_barrier()

        iota = lax.iota(jnp.int32, 16)
        dnums = lax.GatherDimensionNumbers(
            offset_dims=(), collapsed_slice_dims=(0,), start_index_map=(0,))

        def bcast(v, pat):
            return lax.gather(v, pat[:, None], dimension_numbers=dnums,
                              slice_sizes=(1,),
                              mode=lax.GatherScatterMode.PROMISE_IN_BOUNDS)

        pats = [(q * 16 + iota) >> csh for q in range(Q)]

        def load_idx(cc):
            d0 = pltpu.async_copy(ei_ref.at[0, cc], sidx, sems.at[0, 0])
            d1 = pltpu.async_copy(ei_ref.at[1, cc], didx, sems.at[1, 0])
            d0.wait()
            d1.wait()

        def alpha_loop():
            @plsc.parallel_loop(0, _CH, 1, unroll=8)
            def _(i):
                v = ga[i] + gb[i]
                v = jnp.where(v >= 0.0, v, 0.2 * v)
                ga[i] = jnp.exp(v)

        def pass_a(k, _):
            cc = tid + 16 * k

            @pl.when(cc < _NCHUNK)
            def _():
                load_idx(cc)
                ds_ = []
                for j in range(_CR):
                    sl = pl.ds(j * 128, 128)
                    ds_.append(pltpu.async_copy(at_ref.at[sidx.at[j]], ga.at[sl], sems.at[0, j]))
                    ds_.append(pltpu.async_copy(bt_ref.at[didx.at[j]], gb.at[sl], sems.at[1, j]))
                for d in ds_:
                    d.wait()
                alpha_loop()
                mine = (cc >= cid * _NB) & (cc < (cid + 1) * _NB)

                @pl.when(mine)
                def _():
                    pltpu.async_copy(ga, al_ref.at[pl.ds(cc * _CH, _CH)],
                                     sems.at[2, 0]).wait()
                prev = []
                for j in range(_CR):
                    sl = pl.ds(j * 128, 128)
                    cur = [pltpu.async_copy(ga.at[sl], den_s.at[didx.at[j]], ssems.at[j & 1], add=True)]
                    for d in prev:
                        d.wait()
                    prev = cur
                for d in prev:
                    d.wait()
            return 0

        lax.fori_loop(0, (_NCHUNK + 15) // 16, pass_a, 0)
        plsc.subcore_barrier()

        def weight_loop():
            @plsc.parallel_loop(0, _CH, 1, unroll=4)
            def _(i):
                w = ga[i] / gd[i]
                for q in range(Q):
                    sl = pl.ds(q * 16, 16)
                    hg[i, sl] = hg[i, sl] * bcast(w, pats[q])

        def pass_b(k, _):
            cb = tid + 16 * k

            @pl.when(cb < _NB)
            def _():
                cc = cid * _NB + cb
                load_idx(cc)
                ds_ = [pltpu.async_copy(al_ref.at[pl.ds(cc * _CH, _CH)], ga,
                                        sems.at[0, 0])]
                for j in range(_CR):
                    sl = pl.ds(j * 128, 128)
                    ds_.append(pltpu.async_copy(den_s.at[didx.at[j]], gd.at[sl], sems.at[2, j]))
                    ds_.append(pltpu.async_copy(h_ref.at[sidx.at[j]], hg.at[sl], sems.at[3, j]))
                for d in ds_:
                    d.wait()
                weight_loop()
                prev = []
                for j in range(_CR):
                    sl = pl.ds(j * 128, 128)
                    cur = [pltpu.async_copy(hg.at[sl], acc_s.at[didx.at[j]], ssems.at[j & 1], add=True)]
                    for d in prev:
                        d.wait()
                    prev = cur
                for d in prev:
                    d.wait()
            return 0

        lax.fori_loop(0, (_NB + 15) // 16, pass_b, 0)
        plsc.subcore_barrier()
        pltpu.sync_copy(acc_s.at[rows], out_ref.at[cid, rows])

        @pl.when(tid == 15)
        def _():
            pltpu.sync_copy(acc_s.at[tail], out_ref.at[cid, tail])

    return sc


_sc_edge1 = _make_sc_edge(HEADS * NHID, NHID)
_sc_edge2 = _make_sc_edge(NCLASS, NCLASS)


def _edge_phase_jax(h, at, bt, edge_index, heads, ch):
    src = edge_index[0]
    dst = edge_index[1]
    alpha = at[src] + bt[dst]
    alpha = alpha[:, :heads]
    alpha = jax.nn.leaky_relu(alpha, negative_slope=0.2)
    amax = jax.ops.segment_max(alpha, dst, num_segments=N)
    alpha = jnp.exp(alpha - amax[dst])
    denom = jax.ops.segment_sum(alpha, dst, num_segments=N)
    alpha = alpha / (denom[dst] + 1e-16)
    msg = h.reshape(N, heads, ch)[src] * alpha[:, :, None]
    out = jax.ops.segment_sum(msg, dst, num_segments=N)
    return out.reshape(N, heads * ch)


def kernel(data, x, edge_index, W1, a_src1, a_dst1, b1, W2, a_src2, a_dst2, b2):
    eye = jnp.eye(HEADS, dtype=jnp.float32)
    As1 = jnp.pad((eye[:, None, :] * a_src1[:, :, None]).reshape(HEADS * NHID, HEADS),
                  ((0, 0), (0, 16 - HEADS)))
    Ad1 = jnp.pad((eye[:, None, :] * a_dst1[:, :, None]).reshape(HEADS * NHID, HEADS),
                  ((0, 0), (0, 16 - HEADS)))
    As2 = jnp.pad(a_src2.reshape(NCLASS, 1), ((0, 0), (0, 15)))
    Ad2 = jnp.pad(a_dst2.reshape(NCLASS, 1), ((0, 0), (0, 15)))

    ei4 = edge_index.reshape(2, _NCHUNK, _CR, 128)
    z64 = jnp.zeros((N, HEADS * NHID), jnp.float32)
    z16 = jnp.zeros((N, NCLASS), jnp.float32)
    z16 = jnp.zeros((N, 16), jnp.float32)

    h1, at1, bt1 = _tc1(x, W1, As1, Ad1)
    parts1, _ = _sc_edge1(ei4, h1, at1, bt1, z64, z16)
    h2, at2, bt2 = _tc2(parts1, b1.reshape(1, -1), W2, As2, Ad2)
    parts2, _ = _sc_edge2(ei4, h2, at2, bt2, z16, z16)
    return _tc3(parts2, b2.reshape(1, -1))
